# baseline (speedup 1.0000x reference)
import functools
import jax
import jax.numpy as jnp
from jax import lax
from jax.experimental import pallas as pl
from jax.experimental.pallas import tpu as pltpu
from jax.experimental.pallas import tpu_sc as plsc

NC = 2
NS = 16
NW = NC * NS

B, C, H, W = 16, 192, 56, 56
G = 4
ROWS = B * C
RPW = ROWS // NW
K = 4
NCHUNK = RPW // K
LANES = 16
WOFF = (0, 16, 32, 40)


def _compute_chunk(rows_v, out_v, ko):
    def hloop(h, carry):
        for k in range(K):
            for w0 in WOFF:
                a = jnp.maximum(
                    rows_v[G * k, h, pl.ds(w0, LANES)],
                    rows_v[G * k + 1, h, pl.ds(w0, LANES)],
                )
                b2 = jnp.maximum(
                    rows_v[G * k + 2, h, pl.ds(w0, LANES)],
                    rows_v[G * k + 3, h, pl.ds(w0, LANES)],
                )
                out_v[ko + k, h, pl.ds(w0, LANES)] = jnp.maximum(a, b2)
        return carry

    lax.fori_loop(0, H, hloop, 0)


def _body(x_hbm, ridx_hbm, out_hbm, idx_v, rows_v0, rows_v1, out_v, sem0, sem1):
    c = lax.axis_index("c")
    s = lax.axis_index("s")
    w = s * NC + c
    base = w * RPW
    pltpu.sync_copy(ridx_hbm.at[pl.ds(w * RPW * G, RPW * G)], idx_v)

    bufs = (rows_v0, rows_v1)
    sems = (sem0, sem1)

    def gather_start(ci, buf, sem):
        pltpu.async_copy(
            x_hbm.at[idx_v.at[pl.ds(ci * (K * G), K * G)]], buf, sem
        )

    def gather_wait(buf, sem):
        pltpu.make_async_copy(x_hbm.at[idx_v.at[pl.ds(0, K * G)]], buf, sem).wait()

    gather_start(0, rows_v0, sem0)
    gather_start(1, rows_v1, sem1)

    def pair(g, carry):
        ci0 = g * 2
        for bu in range(2):
            ci = ci0 + bu
            gather_wait(bufs[bu], sems[bu])
            _compute_chunk(bufs[bu], out_v, bu * K)

            @pl.when(ci + 2 < NCHUNK)
            def _():
                gather_start(ci + 2, bufs[bu], sems[bu])

        pltpu.sync_copy(out_v, out_hbm.at[pl.ds(base + ci0 * K, 2 * K)])
        return carry

    lax.fori_loop(0, NCHUNK // 2, pair, 0)


@jax.jit
def _run(x3, rid):
    mesh = plsc.VectorSubcoreMesh(core_axis_name="c", subcore_axis_name="s")
    f = functools.partial(
        pl.kernel,
        out_type=jax.ShapeDtypeStruct((ROWS, H, W), jnp.float32),
        mesh=mesh,
        compiler_params=pltpu.CompilerParams(use_tc_tiling_on_sc=False),
        scratch_types=[
            pltpu.VMEM((RPW * G,), jnp.int32),
            pltpu.VMEM((K * G, H, W), jnp.float32),
            pltpu.VMEM((K * G, H, W), jnp.float32),
            pltpu.VMEM((2 * K, H, W), jnp.float32),
            pltpu.SemaphoreType.DMA,
            pltpu.SemaphoreType.DMA,
        ],
    )(_body)
    return f(x3, rid)


def kernel(x, index):
    idx4 = index.astype(jnp.int32) % C
    rid = (
        jnp.arange(B, dtype=jnp.int32)[:, None] * C + idx4[None, :]
    ).reshape(-1)
    x3 = x.reshape(ROWS, H, W)
    outr = _run(x3, rid)
    return outr.reshape(B, C, H, W)


# speedup vs baseline: 2.0704x; 2.0704x over previous
import functools
import jax
import jax.numpy as jnp
from jax import lax
from jax.experimental import pallas as pl
from jax.experimental.pallas import tpu as pltpu
from jax.experimental.pallas import tpu_sc as plsc

NC = 2
NS = 16
NW = NC * NS

B, C, H, W = 16, 192, 56, 56
G = 4
SITES = B * H * W
SB = 8
BLOCKS = SITES // SB
BPW = BLOCKS // NW
NB = 14
NCHUNK = BPW // NB
LANES = 16
OB = C // LANES


def _compute_chunk(colt_v, rows_v, out_v):
    for o in range(OB):
        cols = [colt_v[G * o + j, :] for j in range(G)]
        subs = [jnp.full((LANES,), su, dtype=jnp.int32) for su in range(SB)]

        def bloop(blk, carry, cols=cols, subs=subs, o=o):
            r8 = rows_v.at[blk]
            for su in range(SB):
                g0 = plsc.load_gather(r8, [subs[su], cols[0]])
                g1 = plsc.load_gather(r8, [subs[su], cols[1]])
                g2 = plsc.load_gather(r8, [subs[su], cols[2]])
                g3 = plsc.load_gather(r8, [subs[su], cols[3]])
                out_v[blk, su, pl.ds(o * LANES, LANES)] = jnp.maximum(
                    jnp.maximum(g0, g1), jnp.maximum(g2, g3)
                )
            return carry

        lax.fori_loop(0, NB, bloop, 0)


def _body(x_hbm, colt_hbm, out_hbm, colt_v, rows_v0, rows_v1, out_v, sem0, sem1):
    c = lax.axis_index("c")
    s = lax.axis_index("s")
    w = s * NC + c
    base = w * BPW
    pltpu.sync_copy(colt_hbm, colt_v)

    bufs = (rows_v0, rows_v1)
    sems = (sem0, sem1)

    def load_start(ci, buf, sem):
        pltpu.async_copy(x_hbm.at[pl.ds(base + ci * NB, NB)], buf, sem)

    def load_wait(buf, sem):
        pltpu.make_async_copy(x_hbm.at[pl.ds(base, NB)], buf, sem).wait()

    load_start(0, rows_v0, sem0)
    load_start(1, rows_v1, sem1)

    def pair(g, carry):
        for bu in range(2):
            ci = g * 2 + bu
            load_wait(bufs[bu], sems[bu])
            _compute_chunk(colt_v, bufs[bu], out_v)
            pltpu.sync_copy(out_v, out_hbm.at[pl.ds(base + ci * NB, NB)])

            @pl.when(ci + 2 < NCHUNK)
            def _():
                load_start(ci + 2, bufs[bu], sems[bu])

        return carry

    lax.fori_loop(0, NCHUNK // 2, pair, 0)


@jax.jit
def _run(x3, colt):
    mesh = plsc.VectorSubcoreMesh(core_axis_name="c", subcore_axis_name="s")
    f = functools.partial(
        pl.kernel,
        out_type=jax.ShapeDtypeStruct((BLOCKS, SB, C), jnp.float32),
        mesh=mesh,
        compiler_params=pltpu.CompilerParams(
            use_tc_tiling_on_sc=True, needs_layout_passes=False
        ),
        scratch_types=[
            pltpu.VMEM((G * OB, LANES), jnp.int32),
            pltpu.VMEM((NB, SB, C), jnp.float32),
            pltpu.VMEM((NB, SB, C), jnp.float32),
            pltpu.VMEM((NB, SB, C), jnp.float32),
            pltpu.SemaphoreType.DMA,
            pltpu.SemaphoreType.DMA,
        ],
    )(_body)
    return f(x3, colt)


def kernel(x, index):
    idx4 = (index.astype(jnp.int32) % C).reshape(C, G)
    colt = idx4.reshape(OB, LANES, G).transpose(0, 2, 1).reshape(G * OB, LANES)
    x3 = jnp.transpose(x, (0, 2, 3, 1)).reshape(BLOCKS, SB, C)
    o3 = _run(x3, colt)
    return o3.reshape(B, H, W, C).transpose(0, 3, 1, 2)


# speedup vs baseline: 3.1963x; 1.5438x over previous
import functools
import jax
import jax.numpy as jnp
from jax import lax
from jax.experimental import pallas as pl
from jax.experimental.pallas import tpu as pltpu
from jax.experimental.pallas import tpu_sc as plsc

NC = 2
NS = 16
NW = NC * NS

B, C, H, W = 16, 192, 56, 56
G = 4
SITES = B * H * W
SB = 8
BLOCKS = SITES // SB
BPW = BLOCKS // NW
NB = 14
NCHUNK = BPW // NB
LANES = 16
OB = C // LANES


def _compute_chunk(colt_v, rows_v, out_v):
    for o in range(OB):
        cols = [colt_v[G * o + j, :] for j in range(G)]
        subs = [jnp.full((LANES,), su, dtype=jnp.int32) for su in range(SB)]

        def bloop(blk, carry, cols=cols, subs=subs, o=o):
            r8 = rows_v.at[blk]
            for half in range(SB // 4):
                gs = []
                for su4 in range(4):
                    su = half * 4 + su4
                    gs.append([
                        plsc.load_gather(r8, [subs[su], cols[j]])
                        for j in range(G)
                    ])
                for su4 in range(4):
                    su = half * 4 + su4
                    g0, g1, g2, g3 = gs[su4]
                    out_v[blk, su, pl.ds(o * LANES, LANES)] = jnp.maximum(
                        jnp.maximum(g0, g1), jnp.maximum(g2, g3)
                    )
            return carry

        lax.fori_loop(0, NB, bloop, 0)


def _body(x_hbm, colt_hbm, out_hbm, colt_v, rows_v0, rows_v1, out_v, sem0, sem1):
    c = lax.axis_index("c")
    s = lax.axis_index("s")
    w = s * NC + c
    base = w * BPW
    pltpu.sync_copy(colt_hbm, colt_v)

    bufs = (rows_v0, rows_v1)
    sems = (sem0, sem1)

    def load_start(ci, buf, sem):
        pltpu.async_copy(x_hbm.at[pl.ds(base + ci * NB, NB)], buf, sem)

    def load_wait(buf, sem):
        pltpu.make_async_copy(x_hbm.at[pl.ds(base, NB)], buf, sem).wait()

    load_start(0, rows_v0, sem0)
    load_start(1, rows_v1, sem1)

    def pair(g, carry):
        for bu in range(2):
            ci = g * 2 + bu
            load_wait(bufs[bu], sems[bu])
            _compute_chunk(colt_v, bufs[bu], out_v)
            pltpu.sync_copy(out_v, out_hbm.at[pl.ds(base + ci * NB, NB)])

            @pl.when(ci + 2 < NCHUNK)
            def _():
                load_start(ci + 2, bufs[bu], sems[bu])

        return carry

    lax.fori_loop(0, NCHUNK // 2, pair, 0)


@jax.jit
def _run(x3, colt):
    mesh = plsc.VectorSubcoreMesh(core_axis_name="c", subcore_axis_name="s")
    f = functools.partial(
        pl.kernel,
        out_type=jax.ShapeDtypeStruct((BLOCKS, SB, C), jnp.float32),
        mesh=mesh,
        compiler_params=pltpu.CompilerParams(
            use_tc_tiling_on_sc=True, needs_layout_passes=False
        ),
        scratch_types=[
            pltpu.VMEM((G * OB, LANES), jnp.int32),
            pltpu.VMEM((NB, SB, C), jnp.float32),
            pltpu.VMEM((NB, SB, C), jnp.float32),
            pltpu.VMEM((NB, SB, C), jnp.float32),
            pltpu.SemaphoreType.DMA,
            pltpu.SemaphoreType.DMA,
        ],
    )(_body)
    return f(x3, colt)


def kernel(x, index):
    idx4 = (index.astype(jnp.int32) % C).reshape(C, G)
    colt = idx4.reshape(OB, LANES, G).transpose(0, 2, 1).reshape(G * OB, LANES)
    x3 = jnp.transpose(x, (0, 2, 3, 1)).reshape(BLOCKS, SB, C)
    o3 = _run(x3, colt)
    return o3.reshape(B, H, W, C).transpose(0, 3, 1, 2)


# speedup vs baseline: 3.3931x; 1.0616x over previous
import functools
import jax
import jax.numpy as jnp
from jax import lax
from jax.experimental import pallas as pl
from jax.experimental.pallas import tpu as pltpu
from jax.experimental.pallas import tpu_sc as plsc

NC = 2
NS = 16
NW = NC * NS

B, C, H, W = 16, 192, 56, 56
G = 4
SITES = B * H * W
SB = 8
BLOCKS = SITES // SB
BPW = BLOCKS // NW
NB = 7
NCHUNK = BPW // NB
LANES = 16
OB = C // LANES


def _compute_chunk(colt_v, rows_v, out_v):
    for o in range(OB):
        cols = [colt_v[G * o + j, :] for j in range(G)]
        subs = [jnp.full((LANES,), su, dtype=jnp.int32) for su in range(SB)]

        def bloop(blk, carry, cols=cols, subs=subs, o=o):
            r8 = rows_v.at[blk]
            for half in range(SB // 4):
                gs = []
                for su4 in range(4):
                    su = half * 4 + su4
                    gs.append([
                        plsc.load_gather(r8, [subs[su], cols[j]])
                        for j in range(G)
                    ])
                for su4 in range(4):
                    su = half * 4 + su4
                    g0, g1, g2, g3 = gs[su4]
                    out_v[blk, su, pl.ds(o * LANES, LANES)] = jnp.maximum(
                        jnp.maximum(g0, g1), jnp.maximum(g2, g3)
                    )
            return carry

        lax.fori_loop(0, NB, bloop, 0)


def _body(
    x_hbm, colt_hbm, out_hbm,
    colt_v, rows_v0, rows_v1, out_v0, out_v1,
    isem0, isem1, osem0, osem1,
):
    c = lax.axis_index("c")
    s = lax.axis_index("s")
    w = s * NC + c
    base = w * BPW
    pltpu.sync_copy(colt_hbm, colt_v)

    ibufs = (rows_v0, rows_v1)
    isems = (isem0, isem1)
    obufs = (out_v0, out_v1)
    osems = (osem0, osem1)

    def load_start(ci, buf, sem):
        pltpu.async_copy(x_hbm.at[pl.ds(base + ci * NB, NB)], buf, sem)

    def load_wait(buf, sem):
        pltpu.make_async_copy(x_hbm.at[pl.ds(base, NB)], buf, sem).wait()

    def store_start(ci, buf, sem):
        pltpu.async_copy(buf, out_hbm.at[pl.ds(base + ci * NB, NB)], sem)

    def store_wait(buf, sem):
        pltpu.make_async_copy(buf, out_hbm.at[pl.ds(base, NB)], sem).wait()

    load_start(0, rows_v0, isem0)
    load_start(1, rows_v1, isem1)

    def pair(g, carry):
        for bu in range(2):
            ci = g * 2 + bu
            load_wait(ibufs[bu], isems[bu])

            @pl.when(ci >= 2)
            def _():
                store_wait(obufs[bu], osems[bu])

            _compute_chunk(colt_v, ibufs[bu], obufs[bu])
            store_start(ci, obufs[bu], osems[bu])

            @pl.when(ci + 2 < NCHUNK)
            def _():
                load_start(ci + 2, ibufs[bu], isems[bu])

        return carry

    lax.fori_loop(0, NCHUNK // 2, pair, 0)
    store_wait(out_v0, osem0)
    store_wait(out_v1, osem1)


@jax.jit
def _run(x3, colt):
    mesh = plsc.VectorSubcoreMesh(core_axis_name="c", subcore_axis_name="s")
    f = functools.partial(
        pl.kernel,
        out_type=jax.ShapeDtypeStruct((BLOCKS, SB, C), jnp.float32),
        mesh=mesh,
        compiler_params=pltpu.CompilerParams(
            use_tc_tiling_on_sc=True, needs_layout_passes=False
        ),
        scratch_types=[
            pltpu.VMEM((G * OB, LANES), jnp.int32),
            pltpu.VMEM((NB, SB, C), jnp.float32),
            pltpu.VMEM((NB, SB, C), jnp.float32),
            pltpu.VMEM((NB, SB, C), jnp.float32),
            pltpu.VMEM((NB, SB, C), jnp.float32),
            pltpu.SemaphoreType.DMA,
            pltpu.SemaphoreType.DMA,
            pltpu.SemaphoreType.DMA,
            pltpu.SemaphoreType.DMA,
        ],
    )(_body)
    return f(x3, colt)


def kernel(x, index):
    idx4 = (index.astype(jnp.int32) % C).reshape(C, G)
    colt = idx4.reshape(OB, LANES, G).transpose(0, 2, 1).reshape(G * OB, LANES)
    x3 = jnp.transpose(x, (0, 2, 3, 1)).reshape(BLOCKS, SB, C)
    o3 = _run(x3, colt)
    return o3.reshape(B, H, W, C).transpose(0, 3, 1, 2)


# speedup vs baseline: 3.7679x; 1.1104x over previous
import functools
import jax
import jax.numpy as jnp
from jax import lax
from jax.experimental import pallas as pl
from jax.experimental.pallas import tpu as pltpu
from jax.experimental.pallas import tpu_sc as plsc

NC = 2
NS = 16
NW = NC * NS

B, C, H, W = 16, 192, 56, 56
G = 4
SITES = B * H * W
SB = 8
BLOCKS = SITES // SB
BPW = BLOCKS // NW
NB = 7
NCHUNK = BPW // NB
LANES = 16
OB = C // LANES


def _compute_chunk(colt_v, rows_v, out_v):
    for o in range(OB):
        cols = [colt_v[G * o + j, :] for j in range(G)]
        subs = [jnp.full((LANES,), su, dtype=jnp.int32) for su in range(SB)]

        @plsc.parallel_loop(0, NB)
        def bloop(blk, cols=cols, subs=subs, o=o):
            r8 = rows_v.at[blk]
            for half in range(SB // 4):
                gs = []
                for su4 in range(4):
                    su = half * 4 + su4
                    gs.append([
                        plsc.load_gather(r8, [subs[su], cols[j]])
                        for j in range(G)
                    ])
                for su4 in range(4):
                    su = half * 4 + su4
                    g0, g1, g2, g3 = gs[su4]
                    out_v[blk, su, pl.ds(o * LANES, LANES)] = jnp.maximum(
                        jnp.maximum(g0, g1), jnp.maximum(g2, g3)
                    )


def _body(
    x_hbm, colt_hbm, out_hbm,
    colt_v, rows_v0, rows_v1, out_v0, out_v1,
    isem0, isem1, osem0, osem1,
):
    c = lax.axis_index("c")
    s = lax.axis_index("s")
    w = s * NC + c
    base = w * BPW
    pltpu.sync_copy(colt_hbm, colt_v)

    ibufs = (rows_v0, rows_v1)
    isems = (isem0, isem1)
    obufs = (out_v0, out_v1)
    osems = (osem0, osem1)

    def load_start(ci, buf, sem):
        pltpu.async_copy(x_hbm.at[pl.ds(base + ci * NB, NB)], buf, sem)

    def load_wait(buf, sem):
        pltpu.make_async_copy(x_hbm.at[pl.ds(base, NB)], buf, sem).wait()

    def store_start(ci, buf, sem):
        pltpu.async_copy(buf, out_hbm.at[pl.ds(base + ci * NB, NB)], sem)

    def store_wait(buf, sem):
        pltpu.make_async_copy(buf, out_hbm.at[pl.ds(base, NB)], sem).wait()

    load_start(0, rows_v0, isem0)
    load_start(1, rows_v1, isem1)

    def pair(g, carry):
        for bu in range(2):
            ci = g * 2 + bu
            load_wait(ibufs[bu], isems[bu])

            @pl.when(ci >= 2)
            def _():
                store_wait(obufs[bu], osems[bu])

            _compute_chunk(colt_v, ibufs[bu], obufs[bu])
            store_start(ci, obufs[bu], osems[bu])

            @pl.when(ci + 2 < NCHUNK)
            def _():
                load_start(ci + 2, ibufs[bu], isems[bu])

        return carry

    lax.fori_loop(0, NCHUNK // 2, pair, 0)
    store_wait(out_v0, osem0)
    store_wait(out_v1, osem1)


@jax.jit
def _run(x3, colt):
    mesh = plsc.VectorSubcoreMesh(core_axis_name="c", subcore_axis_name="s")
    f = functools.partial(
        pl.kernel,
        out_type=jax.ShapeDtypeStruct((BLOCKS, SB, C), jnp.float32),
        mesh=mesh,
        compiler_params=pltpu.CompilerParams(
            use_tc_tiling_on_sc=True, needs_layout_passes=False
        ),
        scratch_types=[
            pltpu.VMEM((G * OB, LANES), jnp.int32),
            pltpu.VMEM((NB, SB, C), jnp.float32),
            pltpu.VMEM((NB, SB, C), jnp.float32),
            pltpu.VMEM((NB, SB, C), jnp.float32),
            pltpu.VMEM((NB, SB, C), jnp.float32),
            pltpu.SemaphoreType.DMA,
            pltpu.SemaphoreType.DMA,
            pltpu.SemaphoreType.DMA,
            pltpu.SemaphoreType.DMA,
        ],
    )(_body)
    return f(x3, colt)


def kernel(x, index):
    idx4 = (index.astype(jnp.int32) % C).reshape(C, G)
    colt = idx4.reshape(OB, LANES, G).transpose(0, 2, 1).reshape(G * OB, LANES)
    x3 = jnp.transpose(x, (0, 2, 3, 1)).reshape(BLOCKS, SB, C)
    o3 = _run(x3, colt)
    return o3.reshape(B, H, W, C).transpose(0, 3, 1, 2)
